# baseline (speedup 1.0000x reference)
import functools

import jax
import jax.numpy as jnp
from jax import lax
from jax.experimental import pallas as pl
from jax.experimental.pallas import tpu as pltpu
from jax.experimental.pallas import tpu_sc as plsc

L = 16
NC = 2
NS = 16
NW = NC * NS
P = 128


def _dyn16(src, idx):
    return lax.gather(
        src, idx[:, None],
        lax.GatherDimensionNumbers(
            offset_dims=(), collapsed_slice_dims=(0,), start_index_map=(0,)),
        (1,), mode=lax.GatherScatterMode.PROMISE_IN_BOUNDS)


def _build_sc_interp(B, C, H, W, N):
    pts_total = B * N
    assert pts_total % NW == 0
    ppw = pts_total // NW
    assert ppw % P == 0
    n_chunks = ppw // P
    assert N % ppw == 0
    wpb = N // ppw
    assert W % 4 == 0 and C == 4
    WQ = W // 4
    VR = B * H * WQ

    mesh = plsc.VectorSubcoreMesh(core_axis_name="c", subcore_axis_name="s",
                                  num_cores=NC, num_subcores=NS)

    @functools.partial(
        pl.kernel,
        out_type=jax.ShapeDtypeStruct((B * N * C,), jnp.float32),
        mesh=mesh,
        compiler_params=pltpu.CompilerParams(use_tc_tiling_on_sc=False),
        scratch_types=[
            pltpu.VMEM((P,), jnp.float32),
            pltpu.VMEM((P,), jnp.float32),
            pltpu.VMEM((P,), jnp.float32),
            pltpu.VMEM((P,), jnp.float32),
            pltpu.VMEM((P,), jnp.int32),
            [pltpu.VMEM((P,), jnp.int32) for _ in range(4)],
            [pltpu.VMEM((P, 4 * C), jnp.float32) for _ in range(4)],
            pltpu.VMEM((P * C,), jnp.float32),
            pltpu.SemaphoreType.DMA,
        ],
    )
    def sc_interp(table_hbm, xy_hbm, out_hbm,
                  xs_v, ys_v, wx_v, wy_v, off_v, idx_v, g_v, out_v, sem):
        cid = lax.axis_index("c")
        sid = lax.axis_index("s")
        wid = sid * NC + cid
        b = wid // wpb
        n_base = (wid % wpb) * ppw

        def chunk_body(chunk, carry):
            n0 = n_base + chunk * P
            pltpu.sync_copy(xy_hbm.at[pl.ds(b * 2 * N + n0, P)], xs_v)
            pltpu.sync_copy(xy_hbm.at[pl.ds(b * 2 * N + N + n0, P)], ys_v)

            def phase1(g, c1):
                sl = pl.ds(g * L, L)
                x = xs_v[sl] * float(W - 1)
                y = ys_v[sl] * float(H - 1)
                x0 = jnp.clip(x.astype(jnp.int32), 0, W - 2)
                y0 = jnp.clip(y.astype(jnp.int32), 0, H - 2)
                wx_v[sl] = x - x0.astype(jnp.float32)
                wy_v[sl] = y - y0.astype(jnp.float32)
                off_v[sl] = (x0 & 3) * 4
                r00 = (b * H + y0) * WQ + (x0 >> 2)
                idx_v[0][sl] = r00
                idx_v[1][sl] = jnp.minimum(r00 + 1, VR - 1)
                idx_v[2][sl] = r00 + WQ
                idx_v[3][sl] = jnp.minimum(r00 + WQ + 1, VR - 1)
                return c1

            lax.fori_loop(0, P // L, phase1, 0)

            copies = [pltpu.async_copy(table_hbm.at[idx_v[k]], g_v[k], sem)
                      for k in range(4)]
            for cp in copies:
                cp.wait()

            def phase3(g, c3):
                iota = lax.iota(jnp.int32, L)
                i3 = iota & 3
                grp = lax.shift_right_logical(iota, 2)
                s = g * L
                sl = pl.ds(s, L)
                wx16 = wx_v[sl]
                wy16 = wy_v[sl]
                off16 = off_v[sl]
                accs = [jnp.zeros((L,), jnp.float32) for _ in range(4)]
                for p in range(L):
                    row = s + p
                    r00 = g_v[0][row, :]
                    r01 = g_v[1][row, :]
                    r10 = g_v[2][row, :]
                    r11 = g_v[3][row, :]
                    pb = jnp.full((L,), p, jnp.int32)
                    offb = _dyn16(off16, pb)
                    pidx = offb + i3
                    ltf = (-lax.shift_right_arithmetic(offb - 12, 31)
                           ).astype(jnp.float32)
                    s00 = _dyn16(r00, pidx)
                    s01a = _dyn16(r00, (pidx + 4) & 15)
                    s01b = _dyn16(r01, i3)
                    s01 = s01b + (s01a - s01b) * ltf
                    s10 = _dyn16(r10, pidx)
                    s11a = _dyn16(r10, (pidx + 4) & 15)
                    s11b = _dyn16(r11, i3)
                    s11 = s11b + (s11a - s11b) * ltf
                    wxb = _dyn16(wx16, pb)
                    wyb = _dyn16(wy16, pb)
                    exb = 1.0 - wxb
                    vx0 = s00 * exb + s01 * wxb
                    vx1 = s10 * exb + s11 * wxb
                    outp = vx0 * (1.0 - wyb) + vx1 * wyb
                    mi = 1 - jnp.minimum(jnp.abs(grp - (p & 3)), 1)
                    accs[p // 4] = accs[p // 4] + outp * mi.astype(jnp.float32)
                for j in range(4):
                    out_v[pl.ds((s + j * 4) * C, L)] = accs[j]
                return c3

            lax.fori_loop(0, P // L, phase3, 0)

            pltpu.sync_copy(out_v,
                            out_hbm.at[pl.ds((b * N + n0) * C, P * C)])
            return carry

        lax.fori_loop(0, n_chunks, chunk_body, 0)

    return sc_interp


@jax.jit
def kernel(R, XY_pc):
    B, C, H, W = R.shape
    N = XY_pc.shape[-1]
    eye = jnp.eye(C, dtype=jnp.float32)
    table = jnp.einsum(
        "bcs,ce->bse", R.reshape(B, C, H * W), eye,
        precision=jax.lax.Precision.HIGHEST).reshape(B * H * W // 4, 4 * C)
    xy = XY_pc.reshape(B * 2 * N)
    sc_interp = _build_sc_interp(B, C, H, W, N)
    out = sc_interp(table, xy)
    return jnp.transpose(out.reshape(B, N, C), (0, 2, 1))


# speedup vs baseline: 6.4179x; 6.4179x over previous
import functools

import jax
import jax.numpy as jnp
import numpy as np
from jax import lax
from jax.experimental import pallas as pl
from jax.experimental.pallas import tpu as pltpu
from jax.experimental.pallas import tpu_sc as plsc

L = 16
NC = 2
NS = 16
NW = NC * NS
P = 128


def _dyn16(src, idx):
    return lax.gather(
        src, idx[:, None],
        lax.GatherDimensionNumbers(
            offset_dims=(), collapsed_slice_dims=(0,), start_index_map=(0,)),
        (1,), mode=lax.GatherScatterMode.PROMISE_IN_BOUNDS)


def _build_sc_interp(B, C, H, W, N):
    pts_total = B * N
    assert pts_total % NW == 0
    ppw = pts_total // NW
    assert ppw % P == 0
    n_chunks = ppw // P
    assert N % ppw == 0
    wpb = N // ppw
    assert W % 4 == 0 and C == 4
    WQ = W // 4
    VR = B * H * WQ

    mesh = plsc.VectorSubcoreMesh(core_axis_name="c", subcore_axis_name="s",
                                  num_cores=NC, num_subcores=NS)

    @functools.partial(
        pl.kernel,
        out_type=jax.ShapeDtypeStruct((B * N * C,), jnp.float32),
        mesh=mesh,
        compiler_params=pltpu.CompilerParams(use_tc_tiling_on_sc=False),
        scratch_types=[
            pltpu.VMEM((P,), jnp.float32),
            pltpu.VMEM((P,), jnp.float32),
            pltpu.VMEM((P,), jnp.float32),
            pltpu.VMEM((P,), jnp.float32),
            pltpu.VMEM((P,), jnp.int32),
            [pltpu.VMEM((P,), jnp.int32) for _ in range(4)],
            [pltpu.VMEM((P, 4 * C), jnp.float32) for _ in range(4)],
            pltpu.VMEM((P * C,), jnp.float32),
            pltpu.SemaphoreType.DMA,
        ],
    )
    def sc_interp(table_hbm, xy_hbm, out_hbm,
                  xs_v, ys_v, wx_v, wy_v, off_v, idx_v, g_v, out_v, sem):
        cid = lax.axis_index("c")
        sid = lax.axis_index("s")
        wid = sid * NC + cid
        b = wid // wpb
        n_base = (wid % wpb) * ppw

        def chunk_body(chunk, carry):
            n0 = n_base + chunk * P
            pltpu.sync_copy(xy_hbm.at[pl.ds(b * 2 * N + n0, P)], xs_v)
            pltpu.sync_copy(xy_hbm.at[pl.ds(b * 2 * N + N + n0, P)], ys_v)

            def phase1(g, c1):
                sl = pl.ds(g * L, L)
                x = xs_v[sl] * float(W - 1)
                y = ys_v[sl] * float(H - 1)
                x0 = jnp.clip(x.astype(jnp.int32), 0, W - 2)
                y0 = jnp.clip(y.astype(jnp.int32), 0, H - 2)
                wx_v[sl] = x - x0.astype(jnp.float32)
                wy_v[sl] = y - y0.astype(jnp.float32)
                off_v[sl] = (x0 & 3) * 4
                r00 = (b * H + y0) * WQ + (x0 >> 2)
                idx_v[0][sl] = r00
                idx_v[1][sl] = jnp.minimum(r00 + 1, VR - 1)
                idx_v[2][sl] = r00 + WQ
                idx_v[3][sl] = jnp.minimum(r00 + WQ + 1, VR - 1)
                return c1

            lax.fori_loop(0, P // L, phase1, 0)

            copies = [pltpu.async_copy(table_hbm.at[idx_v[k]], g_v[k], sem)
                      for k in range(4)]
            for cp in copies:
                cp.wait()

            def phase3(g, c3):
                iota = lax.iota(jnp.int32, L)
                i3 = iota & 3
                grp = lax.shift_right_logical(iota, 2)
                s = g * L
                sl = pl.ds(s, L)
                wx16 = wx_v[sl]
                wy16 = wy_v[sl]
                off16 = off_v[sl]
                accs = [jnp.zeros((L,), jnp.float32) for _ in range(4)]
                for p in range(L):
                    row = s + p
                    r00 = g_v[0][row, :]
                    r01 = g_v[1][row, :]
                    r10 = g_v[2][row, :]
                    r11 = g_v[3][row, :]
                    pb = jnp.full((L,), p, jnp.int32)
                    offb = _dyn16(off16, pb)
                    pidx = offb + i3
                    ltf = (-lax.shift_right_arithmetic(offb - 12, 31)
                           ).astype(jnp.float32)
                    s00 = _dyn16(r00, pidx)
                    s01a = _dyn16(r00, (pidx + 4) & 15)
                    s01b = _dyn16(r01, i3)
                    s01 = s01b + (s01a - s01b) * ltf
                    s10 = _dyn16(r10, pidx)
                    s11a = _dyn16(r10, (pidx + 4) & 15)
                    s11b = _dyn16(r11, i3)
                    s11 = s11b + (s11a - s11b) * ltf
                    wxb = _dyn16(wx16, pb)
                    wyb = _dyn16(wy16, pb)
                    exb = 1.0 - wxb
                    vx0 = s00 * exb + s01 * wxb
                    vx1 = s10 * exb + s11 * wxb
                    outp = vx0 * (1.0 - wyb) + vx1 * wyb
                    mi = 1 - jnp.minimum(jnp.abs(grp - (p & 3)), 1)
                    accs[p // 4] = accs[p // 4] + outp * mi.astype(jnp.float32)
                for j in range(4):
                    out_v[pl.ds((s + j * 4) * C, L)] = accs[j]
                return c3

            lax.fori_loop(0, P // L, phase3, 0)

            pltpu.sync_copy(out_v,
                            out_hbm.at[pl.ds((b * N + n0) * C, P * C)])
            return carry

        lax.fori_loop(0, n_chunks, chunk_body, 0)

    return sc_interp


@jax.jit
def kernel(R, XY_pc):
    B, C, H, W = R.shape
    N = XY_pc.shape[-1]
    XB = 128 // C
    T = H * W // XB
    perm = np.zeros((C, XB, C * XB), np.float32)
    for c in range(C):
        for i in range(XB):
            perm[c, i, i * C + c] = 1.0
    table = jnp.einsum(
        "bcti,cie->bte", R.reshape(B, C, T, XB), jnp.asarray(perm),
        precision=jax.lax.Precision.HIGHEST).reshape(B * H * W // 4, 4 * C)
    xy = XY_pc.reshape(B * 2 * N)
    sc_interp = _build_sc_interp(B, C, H, W, N)
    out = sc_interp(table, xy)
    return jnp.transpose(out.reshape(B, N, C), (0, 2, 1))


# speedup vs baseline: 22.8728x; 3.5639x over previous
import functools

import jax
import jax.numpy as jnp
from jax import lax
from jax.experimental import pallas as pl
from jax.experimental.pallas import tpu as pltpu
from jax.experimental.pallas import tpu_sc as plsc

L = 16
NC = 2
NS = 16
NW = NC * NS
P = 128


def _build_sc_interp(B, C, H, W, N):
    pts_total = B * N
    assert pts_total % NW == 0
    ppw = pts_total // NW
    assert ppw % (2 * P) == 0
    n_chunks = ppw // P
    half = n_chunks // 2
    assert N % ppw == 0
    wpb = N // ppw

    mesh = plsc.VectorSubcoreMesh(core_axis_name="c", subcore_axis_name="s",
                                  num_cores=NC, num_subcores=NS)

    @functools.partial(
        pl.kernel,
        out_type=jax.ShapeDtypeStruct((B * C * N,), jnp.float32),
        mesh=mesh,
        scratch_types=[
            pltpu.VMEM((ppw,), jnp.float32),
            pltpu.VMEM((ppw,), jnp.float32),
            pltpu.VMEM((P,), jnp.float32),
            pltpu.VMEM((P,), jnp.float32),
            pltpu.VMEM((P,), jnp.float32),
            pltpu.VMEM((P,), jnp.float32),
            [[[pltpu.VMEM((P,), jnp.int32) for _ in range(4)]
              for _ in range(4)] for _ in range(2)],
            [[[pltpu.VMEM((P,), jnp.float32) for _ in range(4)]
              for _ in range(4)] for _ in range(2)],
            pltpu.VMEM((4 * ppw,), jnp.float32),
            pltpu.SemaphoreType.DMA,
            pltpu.SemaphoreType.DMA,
        ],
    )
    def sc_interp(table_hbm, xy_hbm, out_hbm,
                  xs_v, ys_v, wxa, wya, wxb, wyb,
                  idx_v, g_v, out_v, semA, semB):
        cid = lax.axis_index("c")
        sid = lax.axis_index("s")
        wid = sid * NC + cid
        b = wid // wpb
        n_base = (wid % wpb) * ppw
        HW = H * W
        sems = (semA, semB)
        wxs = (wxa, wxb)
        wys = (wya, wyb)

        pltpu.sync_copy(xy_hbm.at[pl.ds(b * 2 * N + n_base, ppw)], xs_v)
        pltpu.sync_copy(xy_hbm.at[pl.ds(b * 2 * N + N + n_base, ppw)], ys_v)

        def phase1(buf, chunk):
            co = chunk * P
            for g in range(P // L):
                sl = pl.ds(g * L, L)
                s2 = pl.ds(co + g * L, L)
                x = xs_v[s2] * float(W - 1)
                y = ys_v[s2] * float(H - 1)
                x0 = jnp.clip(x.astype(jnp.int32), 0, W - 2)
                y0 = jnp.clip(y.astype(jnp.int32), 0, H - 2)
                wxs[buf][sl] = x - x0.astype(jnp.float32)
                wys[buf][sl] = y - y0.astype(jnp.float32)
                base = (b * C * H + y0) * W + x0
                for c in range(C):
                    fc = base + c * HW
                    idx_v[buf][0][c][sl] = fc
                    idx_v[buf][1][c][sl] = fc + 1
                    idx_v[buf][2][c][sl] = fc + W
                    idx_v[buf][3][c][sl] = fc + W + 1

        def fire(buf):
            for k in range(4):
                for c in range(C):
                    pltpu.async_copy(table_hbm.at[idx_v[buf][k][c]],
                                     g_v[buf][k][c], sems[buf])

        def drain(buf):
            for k in range(4):
                for c in range(C):
                    pltpu.make_async_copy(table_hbm.at[idx_v[buf][k][c]],
                                          g_v[buf][k][c], sems[buf]).wait()

        def phase3(buf, chunk):
            co = chunk * P
            for g in range(P // L):
                sl = pl.ds(g * L, L)
                wx = wxs[buf][sl]
                wy = wys[buf][sl]
                ex = 1.0 - wx
                ey = 1.0 - wy
                w00 = ex * ey
                w01 = wx * ey
                w10 = ex * wy
                w11 = wx * wy
                gb = g_v[buf]
                for c in range(C):
                    out_v[pl.ds(c * ppw + co + g * L, L)] = (
                        gb[0][c][sl] * w00 + gb[1][c][sl] * w01
                        + gb[2][c][sl] * w10 + gb[3][c][sl] * w11)

        phase1(0, 0)
        fire(0)

        def pair_body(i, carry):
            c0 = 2 * i
            phase1(1, c0 + 1)
            fire(1)
            drain(0)
            phase3(0, c0)
            phase1(0, c0 + 2)
            fire(0)
            drain(1)
            phase3(1, c0 + 1)
            return carry

        lax.fori_loop(0, half - 1, pair_body, 0)

        phase1(1, n_chunks - 1)
        fire(1)
        drain(0)
        phase3(0, n_chunks - 2)
        drain(1)
        phase3(1, n_chunks - 1)

        for c in range(C):
            pltpu.sync_copy(
                out_v.at[pl.ds(c * ppw, ppw)],
                out_hbm.at[pl.ds((b * C + c) * N + n_base, ppw)])

    return sc_interp


@jax.jit
def kernel(R, XY_pc):
    B, C, H, W = R.shape
    N = XY_pc.shape[-1]
    table = R.reshape(B * C * H * W)
    xy = XY_pc.reshape(B * 2 * N)
    sc_interp = _build_sc_interp(B, C, H, W, N)
    out = sc_interp(table, xy)
    return out.reshape(B, C, N)
